# Initial kernel scaffold; baseline (speedup 1.0000x reference)
import jax
import jax.numpy as jnp
from jax.experimental import pallas as pl


def kernel(x, edge_index, W, b):
    raise NotImplementedError("write your pallas kernel here")


# speedup vs baseline: 88.0182x; 88.0182x over previous
import functools

import jax
import jax.numpy as jnp
import numpy as np
from jax import lax
from jax.experimental import pallas as pl
from jax.experimental.pallas import tpu as pltpu
from jax.experimental.pallas import tpu_sc as plsc

NC, NS = 2, 16
NW = NC * NS
B = 128


@functools.partial(jax.jit, static_argnums=(2,))
def _sc_degree(dstp, zeros1d, NB):
    NP = zeros1d.shape[0]
    mesh = plsc.VectorSubcoreMesh(core_axis_name="c", subcore_axis_name="s")

    @functools.partial(
        pl.kernel,
        out_type=jax.ShapeDtypeStruct((NW * NP,), jnp.float32),
        mesh=mesh,
        scratch_types=[
            pltpu.VMEM((NB, B), jnp.int32),
            pltpu.VMEM((NP,), jnp.float32),
        ],
        compiler_params=pltpu.CompilerParams(needs_layout_passes=False),
    )
    def k(dst_hbm, zeros_hbm, degp_hbm, dst_v, hist):
        c = lax.axis_index("c")
        s = lax.axis_index("s")
        wid = s * NC + c
        pltpu.sync_copy(zeros_hbm, hist)
        pltpu.sync_copy(dst_hbm.at[wid], dst_v)
        ones16 = jnp.full((16,), 1.0, jnp.float32)

        def body(t, carry):
            j = t // 8
            kk = t % 8
            v = dst_v[j, pl.ds(kk * 16, 16)]
            plsc.addupdate_scatter(hist, [v], ones16)
            return carry

        lax.fori_loop(jnp.int32(0), jnp.int32(NB * 8), body, jnp.int32(0))
        pltpu.sync_copy(hist, degp_hbm.at[pl.ds(wid * NP, NP)])

    return k(dstp, zeros1d)


@functools.partial(jax.jit, static_argnums=(3,))
def _sc_scatter(h2, srcp, dstp, NB):
    NP, D = h2.shape
    rps = NP // NS
    mesh = plsc.VectorSubcoreMesh(core_axis_name="c", subcore_axis_name="s")

    @functools.partial(
        pl.kernel,
        out_type=jax.ShapeDtypeStruct((NC, NP, D), jnp.float32),
        mesh=mesh,
        scratch_types=[
            pltpu.VMEM((NB, B), jnp.int32),
            pltpu.VMEM((NB, B), jnp.int32),
            pltpu.VMEM((B, D), jnp.float32),
            pltpu.VMEM_SHARED((NP, D), jnp.float32),
            pltpu.SemaphoreType.DMA,
        ],
    )
    def k(h2_hbm, src_hbm, dst_hbm, outp_hbm, src_v, dst_v, rows_v, acc, sem):
        c = lax.axis_index("c")
        s = lax.axis_index("s")
        wid = s * NC + c
        r0 = s * rps
        pltpu.sync_copy(h2_hbm.at[pl.ds(r0, rps)], acc.at[pl.ds(r0, rps)])
        pltpu.sync_copy(src_hbm.at[wid], src_v)
        pltpu.sync_copy(dst_hbm.at[wid], dst_v)
        plsc.subcore_barrier()

        def body(j, carry):
            pltpu.async_copy(h2_hbm.at[src_v.at[j]], rows_v, sem).wait()
            pltpu.sync_copy(rows_v, acc.at[dst_v.at[j]], add=True)
            return carry

        lax.fori_loop(jnp.int32(0), jnp.int32(NB), body, jnp.int32(0))
        plsc.subcore_barrier()
        pltpu.sync_copy(acc.at[pl.ds(r0, rps)], outp_hbm.at[c, pl.ds(r0, rps)])

    return k(h2, srcp, dstp)


def _tc_h2(xp, W, degp):
    NP, Din = xp.shape
    Dout = W.shape[1]
    BN = NP // NS

    def kern(x_ref, w_ref, d_ref, h2_ref, dis_ref):
        d = jnp.sum(d_ref[...], axis=0) + 1.0
        dis = lax.rsqrt(d)
        h = jnp.dot(x_ref[...], w_ref[...], preferred_element_type=jnp.float32)
        h2_ref[...] = h * dis
        dis_ref[...] = dis

    z = np.int32(0)
    return pl.pallas_call(
        kern,
        grid=(NP // BN,),
        in_specs=[
            pl.BlockSpec((BN, Din), lambda i: (i, z)),
            pl.BlockSpec((Din, Dout), lambda i: (z, z)),
            pl.BlockSpec((NW, BN, 1), lambda i: (z, i, z)),
        ],
        out_specs=[
            pl.BlockSpec((BN, Dout), lambda i: (i, z)),
            pl.BlockSpec((BN, 1), lambda i: (i, z)),
        ],
        out_shape=[
            jax.ShapeDtypeStruct((NP, Dout), jnp.float32),
            jax.ShapeDtypeStruct((NP, 1), jnp.float32),
        ],
    )(xp, W, degp)


def _tc_combine(p, h2, dis, b, N):
    D = h2.shape[1]
    BN = 1000

    def kern(p_ref, h2_ref, dis_ref, b_ref, o_ref):
        agg = p_ref[0] + p_ref[1] - h2_ref[...]
        o_ref[...] = jnp.maximum(agg * dis_ref[...] + b_ref[...], 0.0)

    z = np.int32(0)
    return pl.pallas_call(
        kern,
        grid=(N // BN,),
        in_specs=[
            pl.BlockSpec((NC, BN, D), lambda i: (z, i, z)),
            pl.BlockSpec((BN, D), lambda i: (i, z)),
            pl.BlockSpec((BN, 1), lambda i: (i, z)),
            pl.BlockSpec((1, D), lambda i: (z, z)),
        ],
        out_specs=pl.BlockSpec((BN, D), lambda i: (i, z)),
        out_shape=jax.ShapeDtypeStruct((N, D), jnp.float32),
    )(p, h2, dis, b.reshape(1, D))


def kernel(x, edge_index, W, b):
    N, Din = x.shape
    E = edge_index.shape[1]
    x = x.astype(jnp.float32)
    W = W.astype(jnp.float32)
    b = b.astype(jnp.float32)

    src = edge_index[0].astype(jnp.int32)
    dst = edge_index[1].astype(jnp.int32)

    EW = E // NW
    NB = -(-EW // B)
    pad = NB * B - EW
    NP = -(-N // (NS * 8)) * (NS * 8) + NS * 8

    srcp = jnp.concatenate(
        [src.reshape(NW, EW), jnp.zeros((NW, pad), jnp.int32)], axis=1
    ).reshape(NW, NB, B)
    dstp = jnp.concatenate(
        [dst.reshape(NW, EW), jnp.full((NW, pad), N, jnp.int32)], axis=1
    ).reshape(NW, NB, B)

    xp = jnp.concatenate([x, jnp.zeros((NP - N, Din), jnp.float32)], axis=0)
    zeros1d = jnp.zeros((NP,), jnp.float32)

    degp = _sc_degree(dstp, zeros1d, NB).reshape(NW, NP, 1)
    h2, dis = _tc_h2(xp, W, degp)
    outp = _sc_scatter(h2, srcp, dstp, NB)
    out = _tc_combine(outp, h2, dis, b, N)
    return out.astype(jnp.float64)


# speedup vs baseline: 99.6418x; 1.1321x over previous
import functools

import jax
import jax.numpy as jnp
import numpy as np
from jax import lax
from jax.experimental import pallas as pl
from jax.experimental.pallas import tpu as pltpu
from jax.experimental.pallas import tpu_sc as plsc

NC, NS = 2, 16
NW = NC * NS
B = 128


@functools.partial(jax.jit, static_argnums=(2,))
def _sc_degree(dstp, zeros1d, NB):
    NP = zeros1d.shape[0]
    mesh = plsc.VectorSubcoreMesh(core_axis_name="c", subcore_axis_name="s")

    @functools.partial(
        pl.kernel,
        out_type=jax.ShapeDtypeStruct((NW * NP,), jnp.float32),
        mesh=mesh,
        scratch_types=[
            pltpu.VMEM((NB, B), jnp.int32),
            pltpu.VMEM((NP,), jnp.float32),
        ],
        compiler_params=pltpu.CompilerParams(needs_layout_passes=False),
    )
    def k(dst_hbm, zeros_hbm, degp_hbm, dst_v, hist):
        c = lax.axis_index("c")
        s = lax.axis_index("s")
        wid = s * NC + c
        pltpu.sync_copy(zeros_hbm, hist)
        pltpu.sync_copy(dst_hbm.at[wid], dst_v)
        ones16 = jnp.full((16,), 1.0, jnp.float32)

        def body(t, carry):
            j = t // 8
            kk = t % 8
            v = dst_v[j, pl.ds(kk * 16, 16)]
            plsc.addupdate_scatter(hist, [v], ones16)
            return carry

        lax.fori_loop(jnp.int32(0), jnp.int32(NB * 8), body, jnp.int32(0))
        pltpu.sync_copy(hist, degp_hbm.at[pl.ds(wid * NP, NP)])

    return k(dstp, zeros1d)


@functools.partial(jax.jit, static_argnums=(2,))
def _sc_scatter(h2, pk, NB):
    NP, D = h2.shape
    rps = NP // NS
    mesh = plsc.VectorSubcoreMesh(core_axis_name="c", subcore_axis_name="s")

    @functools.partial(
        pl.kernel,
        out_type=jax.ShapeDtypeStruct((NC, NP, D), jnp.float32),
        mesh=mesh,
        scratch_types=[
            pltpu.VMEM((NB, B), jnp.int32),
            pltpu.VMEM((2, B), jnp.int32),
            pltpu.VMEM((2, B), jnp.int32),
            pltpu.VMEM((B, D), jnp.float32),
            pltpu.VMEM((B, D), jnp.float32),
            pltpu.VMEM_SHARED((NP, D), jnp.float32),
            pltpu.SemaphoreType.DMA,
            pltpu.SemaphoreType.DMA,
            pltpu.SemaphoreType.DMA,
            pltpu.SemaphoreType.DMA,
        ],
        compiler_params=pltpu.CompilerParams(needs_layout_passes=False),
    )
    def k(h2_hbm, pk_hbm, outp_hbm,
          pk_v, src_r, dst_r, rows0, rows1, acc, sg0, sg1, ss0, ss1):
        c = lax.axis_index("c")
        s = lax.axis_index("s")
        wid = s * NC + c
        r0 = s * rps
        pltpu.sync_copy(h2_hbm.at[pl.ds(r0, rps)], acc.at[pl.ds(r0, rps)])
        pltpu.sync_copy(pk_hbm.at[wid], pk_v)
        plsc.subcore_barrier()

        S0, S1 = np.int32(0), np.int32(1)

        def unpack(j, slot):
            for kk in range(8):
                v = pk_v[j, pl.ds(kk * 16, 16)]
                src_r[slot, pl.ds(kk * 16, 16)] = v >> 14
                dst_r[slot, pl.ds(kk * 16, 16)] = v & 16383

        def gather_start(slot, buf, sem):
            pltpu.async_copy(h2_hbm.at[src_r.at[slot]], buf, sem)

        def gather_wait(slot, buf, sem):
            pltpu.make_async_copy(h2_hbm.at[src_r.at[slot]], buf, sem).wait()

        def scat_start(slot, buf, sem):
            pltpu.async_copy(buf, acc.at[dst_r.at[slot]], sem, add=True)

        def scat_wait(slot, buf, sem):
            pltpu.make_async_copy(buf, acc.at[dst_r.at[slot]], sem).wait()

        def body(t, carry):
            j0 = 2 * t
            j1 = j0 + 1

            @pl.when(t > 0)
            def _reap():
                scat_wait(S0, rows0, ss0)
                scat_wait(S1, rows1, ss1)

            unpack(j0, S0)
            unpack(j1, S1)
            gather_start(S0, rows0, sg0)
            gather_start(S1, rows1, sg1)
            gather_wait(S0, rows0, sg0)
            scat_start(S0, rows0, ss0)
            gather_wait(S1, rows1, sg1)
            scat_start(S1, rows1, ss1)
            return carry

        lax.fori_loop(jnp.int32(0), jnp.int32(NB // 2), body, jnp.int32(0))
        scat_wait(S0, rows0, ss0)
        scat_wait(S1, rows1, ss1)
        plsc.subcore_barrier()
        pltpu.sync_copy(acc.at[pl.ds(r0, rps)], outp_hbm.at[c, pl.ds(r0, rps)])

    return k(h2, pk)


def _tc_h2(xp, W, degp3):
    NP, Din = xp.shape
    Dout = W.shape[1]
    BN = 1024
    NRB = BN // B
    z = np.int32(0)

    def kern(x_ref, w_ref, d_ref, h2_ref, dis_ref):
        d = jnp.sum(d_ref[...], axis=0) + 1.0
        dis = lax.rsqrt(d)
        h = jnp.dot(x_ref[...], w_ref[...], preferred_element_type=jnp.float32)
        eye = (lax.broadcasted_iota(jnp.int32, (B, B), 0)
               == lax.broadcasted_iota(jnp.int32, (B, B), 1)).astype(jnp.float32)
        for q in range(NRB):
            col = lax.dot_general(
                eye, dis[q:q + 1, :], (((1,), (1,)), ((), ())),
                preferred_element_type=jnp.float32)
            h2_ref[q * B:(q + 1) * B, :] = h[q * B:(q + 1) * B, :] * col
            dis_ref[q * B:(q + 1) * B, :] = col

    return pl.pallas_call(
        kern,
        grid=(NP // BN,),
        in_specs=[
            pl.BlockSpec((BN, Din), lambda i: (i, z)),
            pl.BlockSpec((Din, Dout), lambda i: (z, z)),
            pl.BlockSpec((NW, NRB, B), lambda i: (z, i, z)),
        ],
        out_specs=[
            pl.BlockSpec((BN, Dout), lambda i: (i, z)),
            pl.BlockSpec((BN, 1), lambda i: (i, z)),
        ],
        out_shape=[
            jax.ShapeDtypeStruct((NP, Dout), jnp.float32),
            jax.ShapeDtypeStruct((NP, 1), jnp.float32),
        ],
    )(xp, W, degp3)


def _tc_combine(p, h2, dis, b, N):
    D = h2.shape[1]
    BN = 1000
    z = np.int32(0)

    def kern(p_ref, h2_ref, dis_ref, b_ref, o_ref):
        agg = p_ref[0] + p_ref[1] - h2_ref[...]
        o_ref[...] = jnp.maximum(agg * dis_ref[...] + b_ref[...], 0.0)

    return pl.pallas_call(
        kern,
        grid=(N // BN,),
        in_specs=[
            pl.BlockSpec((NC, BN, D), lambda i: (z, i, z)),
            pl.BlockSpec((BN, D), lambda i: (i, z)),
            pl.BlockSpec((BN, 1), lambda i: (i, z)),
            pl.BlockSpec((1, D), lambda i: (z, z)),
        ],
        out_specs=pl.BlockSpec((BN, D), lambda i: (i, z)),
        out_shape=jax.ShapeDtypeStruct((N, D), jnp.float32),
    )(p, h2, dis, b.reshape(1, D))


def kernel(x, edge_index, W, b):
    N, Din = x.shape
    E = edge_index.shape[1]
    x = x.astype(jnp.float32)
    W = W.astype(jnp.float32)
    b = b.astype(jnp.float32)

    src = edge_index[0].astype(jnp.int32)
    dst = edge_index[1].astype(jnp.int32)

    EW = E // NW
    NB = 2 * (-(-EW // (2 * B)))
    pad = NB * B - EW
    NP = (-(-N // (NS * 8)) + 1) * (NS * 8)

    srcp = jnp.concatenate(
        [src.reshape(NW, EW), jnp.zeros((NW, pad), jnp.int32)], axis=1
    ).reshape(NW, NB, B)
    dstp = jnp.concatenate(
        [dst.reshape(NW, EW), jnp.full((NW, pad), N, jnp.int32)], axis=1
    ).reshape(NW, NB, B)
    pk = (srcp << 14) | dstp

    xp = jnp.concatenate([x, jnp.zeros((NP - N, Din), jnp.float32)], axis=0)
    zeros1d = jnp.zeros((NP,), jnp.float32)

    degp3 = _sc_degree(dstp, zeros1d, NB).reshape(NW, NP // B, B)
    h2, dis = _tc_h2(xp, W, degp3)
    outp = _sc_scatter(h2, pk, NB)
    return _tc_combine(outp, h2, dis, b, N).astype(jnp.float64)


# speedup vs baseline: 131.7108x; 1.3218x over previous
import functools

import jax
import jax.numpy as jnp
import numpy as np
from jax import lax
from jax.experimental import pallas as pl
from jax.experimental.pallas import tpu as pltpu
from jax.experimental.pallas import tpu_sc as plsc

NC, NS = 2, 16
NW = NC * NS
B = 128


@functools.partial(jax.jit, static_argnums=(2,))
def _sc_degree(dstp, zeros1d, NB):
    NP = zeros1d.shape[0]
    mesh = plsc.VectorSubcoreMesh(core_axis_name="c", subcore_axis_name="s")

    @functools.partial(
        pl.kernel,
        out_type=jax.ShapeDtypeStruct((NW * NP,), jnp.float32),
        mesh=mesh,
        scratch_types=[
            pltpu.VMEM((NB, B), jnp.int32),
            pltpu.VMEM((NP,), jnp.float32),
        ],
        compiler_params=pltpu.CompilerParams(needs_layout_passes=False),
    )
    def k(dst_hbm, zeros_hbm, degp_hbm, dst_v, hist):
        c = lax.axis_index("c")
        s = lax.axis_index("s")
        wid = s * NC + c
        pltpu.sync_copy(zeros_hbm, hist)
        pltpu.sync_copy(dst_hbm.at[wid], dst_v)
        ones16 = jnp.full((16,), 1.0, jnp.float32)

        def body(t, carry):
            j = t // 8
            kk = t % 8
            v = dst_v[j, pl.ds(kk * 16, 16)]
            plsc.addupdate_scatter(hist, [v], ones16)
            return carry

        lax.fori_loop(jnp.int32(0), jnp.int32(NB * 8), body, jnp.int32(0))
        pltpu.sync_copy(hist, degp_hbm.at[pl.ds(wid * NP, NP)])

    return k(dstp, zeros1d)


@functools.partial(jax.jit, static_argnums=(3,))
def _sc_scatter(h2, srcp, dstp, NB):
    NP, D = h2.shape
    rps = NP // NS
    mesh = plsc.VectorSubcoreMesh(core_axis_name="c", subcore_axis_name="s")

    @functools.partial(
        pl.kernel,
        out_type=jax.ShapeDtypeStruct((NC, NP, D), jnp.float32),
        mesh=mesh,
        scratch_types=[
            pltpu.VMEM((NB, B), jnp.int32),
            pltpu.VMEM((NB, B), jnp.int32),
            pltpu.VMEM((B, D), jnp.float32),
            pltpu.VMEM_SHARED((NP, D), jnp.float32),
            pltpu.SemaphoreType.DMA,
        ],
    )
    def k(h2_hbm, src_hbm, dst_hbm, outp_hbm, src_v, dst_v, rows0, acc, sg0):
        c = lax.axis_index("c")
        s = lax.axis_index("s")
        wid = s * NC + c
        r0 = s * rps
        pltpu.sync_copy(h2_hbm.at[pl.ds(r0, rps)], acc.at[pl.ds(r0, rps)])
        pltpu.sync_copy(src_hbm.at[wid], src_v)
        pltpu.sync_copy(dst_hbm.at[wid], dst_v)
        plsc.subcore_barrier()

        def body(j, carry):
            pltpu.async_copy(h2_hbm.at[src_v.at[j]], rows0, sg0).wait()
            pltpu.sync_copy(rows0, acc.at[dst_v.at[j]], add=True)
            return carry

        lax.fori_loop(jnp.int32(0), jnp.int32(NB), body, jnp.int32(0))
        plsc.subcore_barrier()
        pltpu.sync_copy(acc.at[pl.ds(r0, rps)], outp_hbm.at[c, pl.ds(r0, rps)])

    return k(h2, srcp, dstp)


def _tc_h2(xp, W, degp3):
    NP, Din = xp.shape
    Dout = W.shape[1]
    BN = 1024
    NRB = BN // B
    z = np.int32(0)

    def kern(x_ref, w_ref, d_ref, h2_ref, dis_ref):
        d = jnp.sum(d_ref[...], axis=0) + 1.0
        dis = lax.rsqrt(d)
        h = jnp.dot(x_ref[...], w_ref[...], preferred_element_type=jnp.float32)
        eye = (lax.broadcasted_iota(jnp.int32, (B, B), 0)
               == lax.broadcasted_iota(jnp.int32, (B, B), 1)).astype(jnp.float32)
        for q in range(NRB):
            col = lax.dot_general(
                eye, dis[q:q + 1, :], (((1,), (1,)), ((), ())),
                preferred_element_type=jnp.float32)
            h2_ref[q * B:(q + 1) * B, :] = h[q * B:(q + 1) * B, :] * col
            dis_ref[q * B:(q + 1) * B, :] = col

    return pl.pallas_call(
        kern,
        grid=(NP // BN,),
        in_specs=[
            pl.BlockSpec((BN, Din), lambda i: (i, z)),
            pl.BlockSpec((Din, Dout), lambda i: (z, z)),
            pl.BlockSpec((NW, NRB, B), lambda i: (z, i, z)),
        ],
        out_specs=[
            pl.BlockSpec((BN, Dout), lambda i: (i, z)),
            pl.BlockSpec((BN, 1), lambda i: (i, z)),
        ],
        out_shape=[
            jax.ShapeDtypeStruct((NP, Dout), jnp.float32),
            jax.ShapeDtypeStruct((NP, 1), jnp.float32),
        ],
    )(xp, W, degp3)


def _tc_combine(p, h2, dis, b, N):
    D = h2.shape[1]
    BN = 1000
    z = np.int32(0)

    def kern(p_ref, h2_ref, dis_ref, b_ref, o_ref):
        agg = p_ref[0] + p_ref[1] - h2_ref[...]
        o_ref[...] = jnp.maximum(agg * dis_ref[...] + b_ref[...], 0.0)

    return pl.pallas_call(
        kern,
        grid=(N // BN,),
        in_specs=[
            pl.BlockSpec((NC, BN, D), lambda i: (z, i, z)),
            pl.BlockSpec((BN, D), lambda i: (i, z)),
            pl.BlockSpec((BN, 1), lambda i: (i, z)),
            pl.BlockSpec((1, D), lambda i: (z, z)),
        ],
        out_specs=pl.BlockSpec((BN, D), lambda i: (i, z)),
        out_shape=jax.ShapeDtypeStruct((N, D), jnp.float32),
    )(p, h2, dis, b.reshape(1, D))


def kernel(x, edge_index, W, b):
    N, Din = x.shape
    E = edge_index.shape[1]
    x = x.astype(jnp.float32)
    W = W.astype(jnp.float32)
    b = b.astype(jnp.float32)

    src = edge_index[0].astype(jnp.int32)
    dst = edge_index[1].astype(jnp.int32)

    EW = E // NW
    NB = -(-EW // B)
    pad = NB * B - EW
    NP = (-(-N // (NS * 8)) + 1) * (NS * 8)

    srcp = jnp.concatenate(
        [src.reshape(NW, EW), jnp.zeros((NW, pad), jnp.int32)], axis=1
    ).reshape(NW, NB, B)
    dstp = jnp.concatenate(
        [dst.reshape(NW, EW), jnp.full((NW, pad), N, jnp.int32)], axis=1
    ).reshape(NW, NB, B)

    xp = jnp.concatenate([x, jnp.zeros((NP - N, Din), jnp.float32)], axis=0)
    zeros1d = jnp.zeros((NP,), jnp.float32)

    degp3 = _sc_degree(dstp, zeros1d, NB).reshape(NW, NP // B, B)
    h2, dis = _tc_h2(xp, W, degp3)
    outp = _sc_scatter(h2, srcp, dstp, NB)
    return _tc_combine(outp, h2, dis, b, N).astype(jnp.float64)
